# baseline (speedup 1.0000x reference)
import jax
import jax.numpy as jnp
from jax.experimental import pallas as pl

_K1 = 6
_BATCH = 16384
_C = 1000
_BLK = 1024


def _body(prob_ref, lab_ref, out_ref):
    i = pl.program_id(0)
    x = prob_ref[...]
    lab = lab_ref[0, 0, :]
    neg = jnp.float32(-jnp.inf)

    m = jnp.max(x, axis=1)
    ms = [m]
    for _ in range(_K1 - 1):
        m = jnp.max(jnp.where(x < m[:, None], x, neg), axis=1)
        ms.append(m)

    ones_n = jnp.ones((_C, 8), jnp.bfloat16)
    rs = []
    for mj in ms[:_K1 - 1]:
        ind = jnp.where(x < mj[:, None], 0.0, 1.0).astype(jnp.bfloat16)
        rj = jax.lax.dot_general(ind, ones_n, (((1,), (0,)), ((), ())),
                                 preferred_element_type=jnp.float32)[:, 0]
        rs.append(rj)

    thresh = ms[_K1 - 1]
    pcnt = rs[_K1 - 2]
    for j in range(_K1 - 2, -1, -1):
        cond = rs[j] >= jnp.float32(_K1)
        thresh = jnp.where(cond, ms[j], thresh)
        prev = rs[j - 1] if j > 0 else jnp.zeros_like(pcnt)
        pcnt = jnp.where(cond, prev, pcnt)

    iota = jax.lax.broadcasted_iota(jnp.int32, (_BLK, _C), 1)
    lab_val = jnp.max(jnp.where(iota == lab[:, None], x, neg), axis=1)

    inter = jnp.where(lab_val > thresh, 1.0, 0.0)
    union = pcnt + 1.0 - inter
    iou = inter / union
    part = jnp.sum(iou)

    @pl.when(i == 0)
    def _init():
        out_ref[...] = jnp.zeros((1, 1), jnp.float32)

    out_ref[...] = out_ref[...] + part


@jax.jit
def kernel(prob, label):
    nb = _BATCH // _BLK
    lab3 = label.reshape(nb, 1, _BLK)
    out = pl.pallas_call(
        _body,
        grid=(nb,),
        in_specs=[
            pl.BlockSpec((_BLK, _C), lambda i: (i, 0)),
            pl.BlockSpec((1, 1, _BLK), lambda i: (i, 0, 0)),
        ],
        out_specs=pl.BlockSpec((1, 1), lambda i: (0, 0)),
        out_shape=jax.ShapeDtypeStruct((1, 1), jnp.float32),
    )(prob, lab3)
    return out[0, 0] / jnp.float32(_BATCH)


# speedup vs baseline: 1.0910x; 1.0910x over previous
import jax
import jax.numpy as jnp
from jax.experimental import pallas as pl

_K1 = 6
_BATCH = 16384
_C = 1000
_BLK = 1024


def _body(prob_ref, lab_ref, out_ref):
    i = pl.program_id(0)
    x = prob_ref[...]
    lab = lab_ref[0, 0, :]
    neg = jnp.float32(-jnp.inf)

    m = jnp.max(x, axis=1)
    ms = [m]
    for _ in range(_K1 - 1):
        m = jnp.max(jnp.where(x < m[:, None], x, neg), axis=1)
        ms.append(m)

    ones_n = jnp.ones((_C, 8), jnp.float32)
    rs = []
    for mj in ms[:_K1 - 1]:
        ind = jnp.where(x < mj[:, None], 0.0, 1.0).astype(jnp.float32)
        rj = jax.lax.dot_general(ind, ones_n, (((1,), (0,)), ((), ())),
                                 preferred_element_type=jnp.float32)[:, 0]
        rs.append(rj)

    thresh = ms[_K1 - 1]
    pcnt = rs[_K1 - 2]
    for j in range(_K1 - 2, -1, -1):
        cond = rs[j] >= jnp.float32(_K1)
        thresh = jnp.where(cond, ms[j], thresh)
        prev = rs[j - 1] if j > 0 else jnp.zeros_like(pcnt)
        pcnt = jnp.where(cond, prev, pcnt)

    iota = jax.lax.broadcasted_iota(jnp.int32, (_BLK, _C), 1)
    lab_val = jnp.max(jnp.where(iota == lab[:, None], x, neg), axis=1)

    inter = jnp.where(lab_val > thresh, 1.0, 0.0)
    union = pcnt + 1.0 - inter
    iou = inter / union
    part = jnp.sum(iou)

    @pl.when(i == 0)
    def _init():
        out_ref[...] = jnp.zeros((1, 1), jnp.float32)

    out_ref[...] = out_ref[...] + part


@jax.jit
def kernel(prob, label):
    nb = _BATCH // _BLK
    lab3 = label.reshape(nb, 1, _BLK)
    out = pl.pallas_call(
        _body,
        grid=(nb,),
        in_specs=[
            pl.BlockSpec((_BLK, _C), lambda i: (i, 0)),
            pl.BlockSpec((1, 1, _BLK), lambda i: (i, 0, 0)),
        ],
        out_specs=pl.BlockSpec((1, 1), lambda i: (0, 0)),
        out_shape=jax.ShapeDtypeStruct((1, 1), jnp.float32),
    )(prob, lab3)
    return out[0, 0] / jnp.float32(_BATCH)
